# Initial kernel scaffold; baseline (speedup 1.0000x reference)
import jax
import jax.numpy as jnp
from jax.experimental import pallas as pl


def kernel(rules, relation, table):
    raise NotImplementedError("write your pallas kernel here")


# speedup vs baseline: 257.1359x; 257.1359x over previous
import functools

import jax
import jax.numpy as jnp
from jax import lax
from jax.experimental import pallas as pl
from jax.experimental.pallas import tpu as pltpu, tpu_sc as plsc

LEN_RULES = 100000
PAD_TOKEN = 100000
B = 16384
L = 200
NUM_CORES = 2
NUM_SUBCORES = 16
NW = NUM_CORES * NUM_SUBCORES
ROWS_PER_W = B // NW
CHUNK_ROWS = 128
NCHUNK = ROWS_PER_W // CHUNK_ROWS
GROUPS = CHUNK_ROWS // 16


def _sc_body(rules_hbm, table_hbm, out_hbm, table_v, rules_v, out_v):
    wid = lax.axis_index("s") * NUM_CORES + lax.axis_index("c")
    base_row = wid * ROWS_PER_W

    pltpu.sync_copy(table_hbm, table_v)

    lane_off = lax.iota(jnp.int32, 16) * L

    for c in range(NCHUNK):
        pltpu.sync_copy(
            rules_hbm.at[pl.ds((base_row + c * CHUNK_ROWS) * L, CHUNK_ROWS * L)],
            rules_v,
        )
        for g in range(GROUPS):
            row_idx = lane_off + (g * 16 * L)

            def step(l, acc):
                rv = plsc.load_gather(rules_v, [row_idx + l])
                logit = plsc.load_gather(table_v, [rv])
                t = 1.0 / (1.0 + jnp.exp(logit))
                t = jnp.where(rv == PAD_TOKEN, 1.0, t)
                return acc * t

            acc = lax.fori_loop(0, L, step, jnp.ones((16,), jnp.float32))
            res = jnp.clip(1.0 - acc, 0.0001, 0.99999)
            out_v[pl.ds(c * CHUNK_ROWS + g * 16, 16)] = res

    pltpu.sync_copy(out_v, out_hbm.at[pl.ds(base_row, ROWS_PER_W)])


@functools.partial(jax.jit, static_argnames=())
def kernel(rules, relation, table):
    del relation
    table_flat = table.reshape(-1)
    rules_flat = rules.reshape(-1)
    mesh = plsc.VectorSubcoreMesh(core_axis_name="c", subcore_axis_name="s")
    out = pl.kernel(
        _sc_body,
        out_type=jax.ShapeDtypeStruct((B,), jnp.float32),
        mesh=mesh,
        scratch_types=[
            pltpu.VMEM((LEN_RULES + 1,), jnp.float32),
            pltpu.VMEM((CHUNK_ROWS * L,), jnp.int32),
            pltpu.VMEM((ROWS_PER_W,), jnp.float32),
        ],
        compiler_params=pltpu.CompilerParams(needs_layout_passes=False),
    )(rules_flat, table_flat)
    return out.reshape(B, 1)


# speedup vs baseline: 338.2470x; 1.3154x over previous
import functools

import jax
import jax.numpy as jnp
from jax import lax
from jax.experimental import pallas as pl
from jax.experimental.pallas import tpu as pltpu, tpu_sc as plsc

LEN_RULES = 100000
PAD_TOKEN = 100000
B = 16384
L = 200
NUM_CORES = 2
NUM_SUBCORES = 16
NW = NUM_CORES * NUM_SUBCORES
ROWS_PER_W = B // NW
CHUNK_ROWS = 128
NCHUNK = ROWS_PER_W // CHUNK_ROWS
GROUPS = CHUNK_ROWS // 16


def _sc_body(rules_hbm, table_hbm, out_hbm, table_v, rules_v, out_v):
    wid = lax.axis_index("s") * NUM_CORES + lax.axis_index("c")
    base_row = wid * ROWS_PER_W

    pltpu.sync_copy(table_hbm, table_v)

    lane_off = lax.iota(jnp.int32, 16) * L

    for c in range(NCHUNK):
        pltpu.sync_copy(
            rules_hbm.at[pl.ds((base_row + c * CHUNK_ROWS) * L, CHUNK_ROWS * L)],
            rules_v,
        )
        for g in range(GROUPS):
            row_idx = lane_off + (g * 16 * L)

            def step(l, den):
                rv = plsc.load_gather(rules_v, [row_idx + l])
                logit = plsc.load_gather(table_v, [rv])
                f = 1.0 + jnp.exp(logit)
                f = jnp.where(rv == PAD_TOKEN, 1.0, f)
                return den * f

            den = lax.fori_loop(0, L, step, jnp.ones((16,), jnp.float32),
                                unroll=8)
            res = jnp.clip(1.0 - 1.0 / den, 0.0001, 0.99999)
            out_v[pl.ds(c * CHUNK_ROWS + g * 16, 16)] = res

    pltpu.sync_copy(out_v, out_hbm.at[pl.ds(base_row, ROWS_PER_W)])


@functools.partial(jax.jit, static_argnames=())
def kernel(rules, relation, table):
    del relation
    table_flat = table.reshape(-1)
    rules_flat = rules.reshape(-1)
    mesh = plsc.VectorSubcoreMesh(core_axis_name="c", subcore_axis_name="s")
    out = pl.kernel(
        _sc_body,
        out_type=jax.ShapeDtypeStruct((B,), jnp.float32),
        mesh=mesh,
        scratch_types=[
            pltpu.VMEM((LEN_RULES + 1,), jnp.float32),
            pltpu.VMEM((CHUNK_ROWS * L,), jnp.int32),
            pltpu.VMEM((ROWS_PER_W,), jnp.float32),
        ],
        compiler_params=pltpu.CompilerParams(needs_layout_passes=False),
    )(rules_flat, table_flat)
    return out.reshape(B, 1)
